# baseline (speedup 1.0000x reference)
import functools

import jax
import jax.numpy as jnp
from jax import lax
from jax.experimental import pallas as pl
from jax.experimental.pallas import tpu as pltpu
from jax.experimental.pallas import tpu_sc as plsc

BT = 256
BI = 512


@functools.lru_cache(maxsize=None)
def _make_row_gather(V, D, B):
    info = plsc.get_sparse_core_info()
    NC, NS = info.num_cores, info.num_subcores
    NW = NC * NS
    assert B % NW == 0
    b_per_w = B // NW
    C = min(64, b_per_w)
    assert b_per_w % C == 0 and (b_per_w % 8 == 0)
    n_chunks = b_per_w // C
    mesh = plsc.VectorSubcoreMesh(core_axis_name="c", subcore_axis_name="s")

    @functools.partial(
        pl.kernel,
        mesh=mesh,
        out_type=jax.ShapeDtypeStruct((B, D), jnp.float32),
        scratch_types=[
            pltpu.VMEM((C,), jnp.int32),
            pltpu.VMEM((C, D), jnp.float32),
            pltpu.SemaphoreType.DMA,
        ],
    )
    def gather(table_hbm, idx_hbm, out_hbm, idx_v, rows_v, sem):
        wid = lax.axis_index("s") * NC + lax.axis_index("c")
        base = wid * b_per_w
        for c in range(n_chunks):
            off = base + c * C
            pltpu.sync_copy(idx_hbm.at[pl.ds(off, C)], idx_v)
            pltpu.async_copy(table_hbm.at[idx_v], rows_v, sem).wait()
            pltpu.sync_copy(rows_v, out_hbm.at[pl.ds(off, C)])

    return gather


def _ffn_body(eid_ref, x_ref, w1_ref, w3_ref, w2_ref, out_ref, *, nj):
    j = pl.program_id(1)
    xv = x_ref[...]
    dn = (((1,), (1,)), ((), ()))
    h1 = lax.dot_general(xv, w1_ref[0], dn, preferred_element_type=jnp.float32)
    h3 = lax.dot_general(xv, w3_ref[0], dn, preferred_element_type=jnp.float32)
    act = h1 * jax.nn.sigmoid(h1) * h3
    contrib = lax.dot_general(act, w2_ref[0], dn,
                              preferred_element_type=jnp.float32)

    @pl.when(j == 0)
    def _():
        out_ref[...] = contrib

    @pl.when(j > 0)
    def _():
        out_ref[...] += contrib


def kernel(x, expert_indices, w13, w2):
    T, D = x.shape
    A = expert_indices.shape[1]
    E = w13.shape[0]
    I = w2.shape[2]
    N = T * A
    NJ = I // BI
    MAX_TILES = N // BT + E
    NP = MAX_TILES * BT

    idx_flat = expert_indices.reshape(N).astype(jnp.int32)
    counts = jnp.bincount(idx_flat, length=E).astype(jnp.int32)
    tiles_per_e = (counts + BT - 1) // BT
    cum_tiles = jnp.cumsum(tiles_per_e)
    padded_start = (cum_tiles - tiles_per_e) * BT
    orig_start = jnp.cumsum(counts) - counts
    order = jnp.argsort(idx_flat, stable=True)
    sorted_e = idx_flat[order]
    dest_row = padded_start[sorted_e] + (jnp.arange(N, dtype=jnp.int32)
                                         - orig_start[sorted_e])
    src_tok = jnp.zeros((NP,), jnp.int32).at[dest_row].set(
        (order // A).astype(jnp.int32))
    inv_row = jnp.zeros((N,), jnp.int32).at[order].set(dest_row)
    tile_eid = jnp.minimum(
        jnp.searchsorted(cum_tiles, jnp.arange(MAX_TILES, dtype=jnp.int32),
                         side="right"),
        E - 1).astype(jnp.int32)

    x_s = _make_row_gather(T, D, NP)(x, src_tok)

    grid_spec = pltpu.PrefetchScalarGridSpec(
        num_scalar_prefetch=1,
        grid=(MAX_TILES, NJ),
        in_specs=[
            pl.BlockSpec((BT, D), lambda t, j, eid: (t, 0)),
            pl.BlockSpec((1, BI, D), lambda t, j, eid: (eid[t], j, 0)),
            pl.BlockSpec((1, BI, D), lambda t, j, eid: (eid[t], NJ + j, 0)),
            pl.BlockSpec((1, D, BI), lambda t, j, eid: (eid[t], 0, j)),
        ],
        out_specs=pl.BlockSpec((BT, D), lambda t, j, eid: (t, 0)),
    )
    y_s = pl.pallas_call(
        functools.partial(_ffn_body, nj=NJ),
        grid_spec=grid_spec,
        out_shape=jax.ShapeDtypeStruct((NP, D), jnp.float32),
        compiler_params=pltpu.CompilerParams(
            dimension_semantics=("arbitrary", "arbitrary")),
    )(tile_eid, x_s, w13, w13, w2)

    out_flat = _make_row_gather(NP, D, N)(y_s, inv_row)
    return out_flat.reshape(T, A, D)


# speedup vs baseline: 1.3974x; 1.3974x over previous
import functools

import jax
import jax.numpy as jnp
from jax import lax
from jax.experimental import pallas as pl
from jax.experimental.pallas import tpu as pltpu
from jax.experimental.pallas import tpu_sc as plsc

BT = 256
BI = 512


@functools.lru_cache(maxsize=None)
def _make_row_gather(V, D, B):
    info = plsc.get_sparse_core_info()
    NC, NS = info.num_cores, info.num_subcores
    NW = NC * NS
    assert B % NW == 0
    b_per_w = B // NW
    C = min(64, b_per_w)
    assert b_per_w % C == 0 and (b_per_w % 8 == 0)
    n_chunks = b_per_w // C
    mesh = plsc.VectorSubcoreMesh(core_axis_name="c", subcore_axis_name="s")

    @functools.partial(
        pl.kernel,
        mesh=mesh,
        out_type=jax.ShapeDtypeStruct((B, D), jnp.float32),
        scratch_types=[
            pltpu.VMEM((C,), jnp.int32),
            pltpu.VMEM((C, D), jnp.float32),
            pltpu.SemaphoreType.DMA,
        ],
    )
    def gather(table_hbm, idx_hbm, out_hbm, idx_v, rows_v, sem):
        wid = lax.axis_index("s") * NC + lax.axis_index("c")
        base = wid * b_per_w
        for c in range(n_chunks):
            off = base + c * C
            pltpu.sync_copy(idx_hbm.at[pl.ds(off, C)], idx_v)
            pltpu.async_copy(table_hbm.at[idx_v], rows_v, sem).wait()
            pltpu.sync_copy(rows_v, out_hbm.at[pl.ds(off, C)])

    return gather


def _ffn_body(eid_ref, nt_ref, x_ref, w1_ref, w3_ref, w2_ref, out_ref, *, nj):
    t = pl.program_id(0)
    j = pl.program_id(1)

    @pl.when(t < nt_ref[0])
    def _():
        xv = x_ref[...]
        dn = (((1,), (1,)), ((), ()))
        h1 = lax.dot_general(xv, w1_ref[0], dn,
                             preferred_element_type=jnp.float32)
        h3 = lax.dot_general(xv, w3_ref[0], dn,
                             preferred_element_type=jnp.float32)
        act = h1 * jax.nn.sigmoid(h1) * h3
        contrib = lax.dot_general(act, w2_ref[0], dn,
                                  preferred_element_type=jnp.float32)

        @pl.when(j == 0)
        def _():
            out_ref[...] = contrib

        @pl.when(j > 0)
        def _():
            out_ref[...] += contrib


def kernel(x, expert_indices, w13, w2):
    T, D = x.shape
    A = expert_indices.shape[1]
    E = w13.shape[0]
    I = w2.shape[2]
    N = T * A
    NJ = I // BI
    MAX_TILES = N // BT + E
    NP = MAX_TILES * BT

    idx_flat = expert_indices.reshape(N).astype(jnp.int32)
    counts = jnp.bincount(idx_flat, length=E).astype(jnp.int32)
    tiles_per_e = (counts + BT - 1) // BT
    cum_tiles = jnp.cumsum(tiles_per_e)
    padded_start = (cum_tiles - tiles_per_e) * BT
    orig_start = jnp.cumsum(counts) - counts
    order = jnp.argsort(idx_flat, stable=True)
    sorted_e = idx_flat[order]
    dest_row = padded_start[sorted_e] + (jnp.arange(N, dtype=jnp.int32)
                                         - orig_start[sorted_e])
    src_tok = (jnp.arange(NP, dtype=jnp.int32) % T).at[dest_row].set(
        (order // A).astype(jnp.int32))
    inv_row = jnp.zeros((N,), jnp.int32).at[order].set(dest_row)
    tile_eid = jnp.minimum(
        jnp.searchsorted(cum_tiles, jnp.arange(MAX_TILES, dtype=jnp.int32),
                         side="right"),
        E - 1).astype(jnp.int32)
    num_tiles = cum_tiles[-1].astype(jnp.int32).reshape(1)

    x_s = _make_row_gather(T, D, NP)(x, src_tok)

    def _xmap(t, j, eid, nt):
        return (jnp.minimum(t, nt[0] - 1), 0)

    def _w1map(t, j, eid, nt):
        tt = jnp.minimum(t, nt[0] - 1)
        return (eid[tt], jnp.where(t < nt[0], j, NJ - 1), 0)

    def _w3map(t, j, eid, nt):
        tt = jnp.minimum(t, nt[0] - 1)
        return (eid[tt], NJ + jnp.where(t < nt[0], j, NJ - 1), 0)

    def _w2map(t, j, eid, nt):
        tt = jnp.minimum(t, nt[0] - 1)
        return (eid[tt], 0, jnp.where(t < nt[0], j, NJ - 1))

    grid_spec = pltpu.PrefetchScalarGridSpec(
        num_scalar_prefetch=2,
        grid=(MAX_TILES, NJ),
        in_specs=[
            pl.BlockSpec((BT, D), _xmap),
            pl.BlockSpec((1, BI, D), _w1map),
            pl.BlockSpec((1, BI, D), _w3map),
            pl.BlockSpec((1, D, BI), _w2map),
        ],
        out_specs=pl.BlockSpec((BT, D), _xmap),
    )
    y_s = pl.pallas_call(
        functools.partial(_ffn_body, nj=NJ),
        grid_spec=grid_spec,
        out_shape=jax.ShapeDtypeStruct((NP, D), jnp.float32),
        compiler_params=pltpu.CompilerParams(
            dimension_semantics=("arbitrary", "arbitrary")),
    )(tile_eid, num_tiles, x_s, w13, w13, w2)

    out_flat = _make_row_gather(NP, D, N)(y_s, inv_row)
    return out_flat.reshape(T, A, D)


# speedup vs baseline: 1.4117x; 1.0102x over previous
import functools

import jax
import jax.numpy as jnp
from jax import lax
from jax.experimental import pallas as pl
from jax.experimental.pallas import tpu as pltpu
from jax.experimental.pallas import tpu_sc as plsc

BT = 256
BI = 512


@functools.lru_cache(maxsize=None)
def _make_row_gather(V, D, B):
    info = plsc.get_sparse_core_info()
    NC, NS = info.num_cores, info.num_subcores
    NW = NC * NS
    assert B % NW == 0
    b_per_w = B // NW
    C = min(64, b_per_w)
    assert b_per_w % C == 0 and (b_per_w % 8 == 0)
    n_chunks = b_per_w // C
    mesh = plsc.VectorSubcoreMesh(core_axis_name="c", subcore_axis_name="s")

    @functools.partial(
        pl.kernel,
        mesh=mesh,
        out_type=jax.ShapeDtypeStruct((B, D), jnp.float32),
        scratch_types=[
            pltpu.VMEM((C,), jnp.int32),
            pltpu.VMEM((C, D), jnp.float32),
            pltpu.SemaphoreType.DMA,
        ],
    )
    def gather(table_hbm, idx_hbm, out_hbm, idx_v, rows_v, sem):
        wid = lax.axis_index("s") * NC + lax.axis_index("c")
        base = wid * b_per_w
        for c in range(n_chunks):
            off = base + c * C
            pltpu.sync_copy(idx_hbm.at[pl.ds(off, C)], idx_v)
            pltpu.async_copy(table_hbm.at[idx_v], rows_v, sem).wait()
            pltpu.sync_copy(rows_v, out_hbm.at[pl.ds(off, C)])

    return gather


def _ffn_body(eid_ref, nt_ref, x_ref, w1_ref, w3_ref, w2_ref, out_ref,
              acc_ref, *, nj):
    j = pl.program_id(0)
    t = pl.program_id(1)

    @pl.when(t < nt_ref[0])
    def _():
        xv = x_ref[...].astype(jnp.bfloat16)
        w1 = w1_ref[0].astype(jnp.bfloat16)
        w3 = w3_ref[0].astype(jnp.bfloat16)
        w2c = w2_ref[0].astype(jnp.bfloat16)
        dn = (((1,), (1,)), ((), ()))
        h1 = lax.dot_general(xv, w1, dn, preferred_element_type=jnp.float32)
        h3 = lax.dot_general(xv, w3, dn, preferred_element_type=jnp.float32)
        act = (h1 * jax.nn.sigmoid(h1) * h3).astype(jnp.bfloat16)
        contrib = lax.dot_general(act, w2c, dn,
                                  preferred_element_type=jnp.float32)

        @pl.when(j == 0)
        def _():
            acc_ref[t] = contrib

        @pl.when(j > 0)
        def _():
            acc_ref[t] = acc_ref[t] + contrib

        @pl.when(j == nj - 1)
        def _():
            out_ref[...] = acc_ref[t]


def kernel(x, expert_indices, w13, w2):
    T, D = x.shape
    A = expert_indices.shape[1]
    E = w13.shape[0]
    I = w2.shape[2]
    N = T * A
    NJ = I // BI
    MAX_TILES = N // BT + E
    NP = MAX_TILES * BT

    idx_flat = expert_indices.reshape(N).astype(jnp.int32)
    counts = jnp.bincount(idx_flat, length=E).astype(jnp.int32)
    tiles_per_e = (counts + BT - 1) // BT
    cum_tiles = jnp.cumsum(tiles_per_e)
    padded_start = (cum_tiles - tiles_per_e) * BT
    orig_start = jnp.cumsum(counts) - counts
    order = jnp.argsort(idx_flat, stable=True)
    sorted_e = idx_flat[order]
    dest_row = padded_start[sorted_e] + (jnp.arange(N, dtype=jnp.int32)
                                         - orig_start[sorted_e])
    src_tok = (jnp.arange(NP, dtype=jnp.int32) % T).at[dest_row].set(
        (order // A).astype(jnp.int32))
    inv_row = jnp.zeros((N,), jnp.int32).at[order].set(dest_row)
    tile_eid = jnp.minimum(
        jnp.searchsorted(cum_tiles, jnp.arange(MAX_TILES, dtype=jnp.int32),
                         side="right"),
        E - 1).astype(jnp.int32)
    num_tiles = cum_tiles[-1].astype(jnp.int32).reshape(1)

    x_s = _make_row_gather(T, D, NP)(x, src_tok)

    def _xmap(j, t, eid, nt):
        return (jnp.minimum(t, nt[0] - 1), 0)

    def _w1map(j, t, eid, nt):
        return (eid[jnp.minimum(t, nt[0] - 1)], j, 0)

    def _w3map(j, t, eid, nt):
        return (eid[jnp.minimum(t, nt[0] - 1)], NJ + j, 0)

    def _w2map(j, t, eid, nt):
        return (eid[jnp.minimum(t, nt[0] - 1)], 0, j)

    def _omap(j, t, eid, nt):
        return (jnp.where(j == NJ - 1, jnp.minimum(t, nt[0] - 1), 0), 0)

    grid_spec = pltpu.PrefetchScalarGridSpec(
        num_scalar_prefetch=2,
        grid=(NJ, MAX_TILES),
        in_specs=[
            pl.BlockSpec((BT, D), _xmap),
            pl.BlockSpec((1, BI, D), _w1map),
            pl.BlockSpec((1, BI, D), _w3map),
            pl.BlockSpec((1, D, BI), _w2map),
        ],
        out_specs=pl.BlockSpec((BT, D), _omap),
        scratch_shapes=[pltpu.VMEM((MAX_TILES, BT, D), jnp.float32)],
    )
    y_s = pl.pallas_call(
        functools.partial(_ffn_body, nj=NJ),
        grid_spec=grid_spec,
        out_shape=jax.ShapeDtypeStruct((NP, D), jnp.float32),
        compiler_params=pltpu.CompilerParams(
            dimension_semantics=("arbitrary", "arbitrary")),
    )(tile_eid, num_tiles, x_s, w13, w13, w2)

    out_flat = _make_row_gather(NP, D, N)(y_s, inv_row)
    return out_flat.reshape(T, A, D)


# speedup vs baseline: 1.7594x; 1.2463x over previous
import functools

import jax
import jax.numpy as jnp
from jax import lax
from jax.experimental import pallas as pl
from jax.experimental.pallas import tpu as pltpu
from jax.experimental.pallas import tpu_sc as plsc

BT = 256
BI = 512


@functools.lru_cache(maxsize=None)
def _make_row_gather(V, D, B):
    info = plsc.get_sparse_core_info()
    NC, NS = info.num_cores, info.num_subcores
    NW = NC * NS
    assert B % NW == 0
    b_per_w = B // NW
    C = min(64, b_per_w)
    assert b_per_w % C == 0 and (b_per_w % 8 == 0)
    n_chunks = b_per_w // C
    mesh = plsc.VectorSubcoreMesh(core_axis_name="c", subcore_axis_name="s")

    @functools.partial(
        pl.kernel,
        mesh=mesh,
        out_type=jax.ShapeDtypeStruct((B, D), jnp.float32),
        scratch_types=[
            pltpu.VMEM((C,), jnp.int32),
            pltpu.VMEM((C, D), jnp.float32),
            pltpu.SemaphoreType.DMA,
        ],
    )
    def gather(table_hbm, idx_hbm, out_hbm, idx_v, rows_v, sem):
        wid = lax.axis_index("s") * NC + lax.axis_index("c")
        base = wid * b_per_w
        for c in range(n_chunks):
            off = base + c * C
            pltpu.sync_copy(idx_hbm.at[pl.ds(off, C)], idx_v)
            pltpu.async_copy(table_hbm.at[idx_v], rows_v, sem).wait()
            pltpu.sync_copy(rows_v, out_hbm.at[pl.ds(off, C)])

    return gather


def _ffn_body(eid_ref, nt_ref, x_ref, w1_ref, w3_ref, w2_ref, out_ref):
    t = pl.program_id(0)

    @pl.when(t < nt_ref[0])
    def _():
        xv = x_ref[...].astype(jnp.bfloat16)
        w1 = w1_ref[0].astype(jnp.bfloat16)
        w3 = w3_ref[0].astype(jnp.bfloat16)
        w2c = w2_ref[0].astype(jnp.bfloat16)
        dn = (((1,), (1,)), ((), ()))
        h1 = lax.dot_general(xv, w1, dn, preferred_element_type=jnp.float32)
        h3 = lax.dot_general(xv, w3, dn, preferred_element_type=jnp.float32)
        act = (h1 * jax.nn.sigmoid(h1) * h3).astype(jnp.bfloat16)
        out_ref[...] = lax.dot_general(act, w2c, dn,
                                       preferred_element_type=jnp.float32)


def kernel(x, expert_indices, w13, w2):
    T, D = x.shape
    A = expert_indices.shape[1]
    E = w13.shape[0]
    I = w2.shape[2]
    N = T * A
    NJ = I // BI
    MAX_TILES = N // BT + E
    NP = MAX_TILES * BT

    idx_flat = expert_indices.reshape(N).astype(jnp.int32)
    counts = jnp.bincount(idx_flat, length=E).astype(jnp.int32)
    tiles_per_e = (counts + BT - 1) // BT
    cum_tiles = jnp.cumsum(tiles_per_e)
    padded_start = (cum_tiles - tiles_per_e) * BT
    orig_start = jnp.cumsum(counts) - counts
    order = jnp.argsort(idx_flat, stable=True)
    sorted_e = idx_flat[order]
    dest_row = padded_start[sorted_e] + (jnp.arange(N, dtype=jnp.int32)
                                         - orig_start[sorted_e])
    src_tok = (jnp.arange(NP, dtype=jnp.int32) % T).at[dest_row].set(
        (order // A).astype(jnp.int32))
    inv_row = jnp.zeros((N,), jnp.int32).at[order].set(dest_row)
    tile_eid = jnp.minimum(
        jnp.searchsorted(cum_tiles, jnp.arange(MAX_TILES, dtype=jnp.int32),
                         side="right"),
        E - 1).astype(jnp.int32)
    num_tiles = cum_tiles[-1].astype(jnp.int32).reshape(1)

    x_s = _make_row_gather(T, D, NP)(x, src_tok)

    def _xmap(t, eid, nt):
        return (jnp.minimum(t, nt[0] - 1), 0)

    def _w1map(t, eid, nt):
        return (eid[jnp.minimum(t, nt[0] - 1)], 0, 0)

    def _w3map(t, eid, nt):
        return (eid[jnp.minimum(t, nt[0] - 1)], 1, 0)

    def _w2map(t, eid, nt):
        return (eid[jnp.minimum(t, nt[0] - 1)], 0, 0)

    grid_spec = pltpu.PrefetchScalarGridSpec(
        num_scalar_prefetch=2,
        grid=(MAX_TILES,),
        in_specs=[
            pl.BlockSpec((BT, D), _xmap),
            pl.BlockSpec((1, I, D), _w1map),
            pl.BlockSpec((1, I, D), _w3map),
            pl.BlockSpec((1, D, I), _w2map),
        ],
        out_specs=pl.BlockSpec((BT, D), _xmap),
    )
    y_s = pl.pallas_call(
        _ffn_body,
        grid_spec=grid_spec,
        out_shape=jax.ShapeDtypeStruct((NP, D), jnp.float32),
        compiler_params=pltpu.CompilerParams(
            dimension_semantics=("arbitrary",)),
    )(tile_eid, num_tiles, x_s, w13, w13, w2)

    out_flat = _make_row_gather(NP, D, N)(y_s, inv_row)
    return out_flat.reshape(T, A, D)


# speedup vs baseline: 1.7683x; 1.0051x over previous
import functools

import jax
import jax.numpy as jnp
from jax import lax
from jax.experimental import pallas as pl
from jax.experimental.pallas import tpu as pltpu
from jax.experimental.pallas import tpu_sc as plsc

BT = 256
BI = 512


@functools.lru_cache(maxsize=None)
def _make_row_gather(V, D, B):
    info = plsc.get_sparse_core_info()
    NC, NS = info.num_cores, info.num_subcores
    NW = NC * NS
    assert B % NW == 0
    b_per_w = B // NW
    C = min(64, b_per_w)
    assert b_per_w % C == 0 and (b_per_w % 8 == 0)
    n_chunks = b_per_w // C
    mesh = plsc.VectorSubcoreMesh(core_axis_name="c", subcore_axis_name="s")

    @functools.partial(
        pl.kernel,
        mesh=mesh,
        out_type=jax.ShapeDtypeStruct((B, D), jnp.float32),
        scratch_types=[
            pltpu.VMEM((C,), jnp.int32),
            pltpu.VMEM((C, D), jnp.float32),
            pltpu.SemaphoreType.DMA,
        ],
    )
    def gather(table_hbm, idx_hbm, out_hbm, idx_v, rows_v, sem):
        wid = lax.axis_index("s") * NC + lax.axis_index("c")
        base = wid * b_per_w
        for c in range(n_chunks):
            off = base + c * C
            pltpu.sync_copy(idx_hbm.at[pl.ds(off, C)], idx_v)
            pltpu.async_copy(table_hbm.at[idx_v], rows_v, sem).wait()
            pltpu.sync_copy(rows_v, out_hbm.at[pl.ds(off, C)])

    return gather


def _ffn_body(eid_ref, nt_ref, x_ref, w1_ref, w3_ref, w2_ref, out_ref):
    t = pl.program_id(0)

    @pl.when(t < nt_ref[0])
    def _():
        xv = x_ref[...]
        w1 = w1_ref[0]
        w3 = w3_ref[0]
        w2c = w2_ref[0]
        dn = (((1,), (1,)), ((), ()))
        h1 = lax.dot_general(xv, w1, dn, preferred_element_type=jnp.float32)
        h3 = lax.dot_general(xv, w3, dn, preferred_element_type=jnp.float32)
        act = h1 * jax.nn.sigmoid(h1) * h3
        out_ref[...] = lax.dot_general(act, w2c, dn,
                                       preferred_element_type=jnp.float32)


def kernel(x, expert_indices, w13, w2):
    T, D = x.shape
    A = expert_indices.shape[1]
    E = w13.shape[0]
    I = w2.shape[2]
    N = T * A
    NJ = I // BI
    MAX_TILES = N // BT + E
    NP = MAX_TILES * BT

    idx_flat = expert_indices.reshape(N).astype(jnp.int32)
    counts = jnp.bincount(idx_flat, length=E).astype(jnp.int32)
    tiles_per_e = (counts + BT - 1) // BT
    cum_tiles = jnp.cumsum(tiles_per_e)
    padded_start = (cum_tiles - tiles_per_e) * BT
    orig_start = jnp.cumsum(counts) - counts
    order = jnp.argsort(idx_flat, stable=True)
    sorted_e = idx_flat[order]
    dest_row = padded_start[sorted_e] + (jnp.arange(N, dtype=jnp.int32)
                                         - orig_start[sorted_e])
    src_tok = (jnp.arange(NP, dtype=jnp.int32) % T).at[dest_row].set(
        (order // A).astype(jnp.int32))
    inv_row = jnp.zeros((N,), jnp.int32).at[order].set(dest_row)
    tile_eid = jnp.minimum(
        jnp.searchsorted(cum_tiles, jnp.arange(MAX_TILES, dtype=jnp.int32),
                         side="right"),
        E - 1).astype(jnp.int32)
    num_tiles = cum_tiles[-1].astype(jnp.int32).reshape(1)

    x_s = _make_row_gather(T, D, NP)(x, src_tok)

    def _xmap(t, eid, nt):
        return (jnp.minimum(t, nt[0] - 1), 0)

    def _w1map(t, eid, nt):
        return (eid[jnp.minimum(t, nt[0] - 1)], 0, 0)

    def _w3map(t, eid, nt):
        return (eid[jnp.minimum(t, nt[0] - 1)], 1, 0)

    def _w2map(t, eid, nt):
        return (eid[jnp.minimum(t, nt[0] - 1)], 0, 0)

    grid_spec = pltpu.PrefetchScalarGridSpec(
        num_scalar_prefetch=2,
        grid=(MAX_TILES,),
        in_specs=[
            pl.BlockSpec((BT, D), _xmap),
            pl.BlockSpec((1, I, D), _w1map),
            pl.BlockSpec((1, I, D), _w3map),
            pl.BlockSpec((1, D, I), _w2map),
        ],
        out_specs=pl.BlockSpec((BT, D), _xmap),
    )
    y_s = pl.pallas_call(
        _ffn_body,
        grid_spec=grid_spec,
        out_shape=jax.ShapeDtypeStruct((NP, D), jnp.float32),
        compiler_params=pltpu.CompilerParams(
            dimension_semantics=("arbitrary",)),
    )(tile_eid, num_tiles, x_s, w13, w13, w2)

    out_flat = _make_row_gather(NP, D, N)(y_s, inv_row)
    return out_flat.reshape(T, A, D)
